# Initial kernel scaffold; baseline (speedup 1.0000x reference)
import functools

import jax
import jax.numpy as jnp
from jax import lax
from jax.experimental import pallas as pl
from jax.experimental.pallas import tpu as pltpu
from jax.experimental.pallas import tpu_sc as plsc

MAX_D = 4
B, S = 4, 2048
N = B * S * S
NUM_WORKERS = 32
PER_WORKER = N // NUM_WORKERS
CHUNK = 16384
NUM_CHUNKS = PER_WORKER // CHUNK
LANES = 16

_mesh = plsc.VectorSubcoreMesh(core_axis_name="c", subcore_axis_name="s")


@functools.partial(
    pl.kernel,
    mesh=_mesh,
    out_type=jax.ShapeDtypeStruct((N,), jnp.float32),
    scratch_types=[
        pltpu.VMEM((LANES,), jnp.float32),
        pltpu.VMEM((CHUNK,), jnp.int32),
        pltpu.VMEM((CHUNK,), jnp.float32),
    ],
)
def _distance_bias_sc(d_hbm, bias_hbm, out_hbm, tab_v, idx_v, out_v):
    wid = lax.axis_index("s") * 2 + lax.axis_index("c")
    pltpu.sync_copy(bias_hbm, tab_v)
    base = wid * PER_WORKER

    def chunk_body(ci, carry):
        off = base + ci * CHUNK
        pltpu.sync_copy(d_hbm.at[pl.ds(off, CHUNK)], idx_v)

        def vec_body(vi, c):
            idx = idx_v[pl.ds(vi * LANES, LANES)]
            idx = jnp.minimum(jnp.maximum(idx, 0), MAX_D)
            out_v[pl.ds(vi * LANES, LANES)] = plsc.load_gather(tab_v, [idx])
            return c

        lax.fori_loop(0, CHUNK // LANES, vec_body, 0)
        pltpu.sync_copy(out_v, out_hbm.at[pl.ds(off, CHUNK)])
        return carry

    lax.fori_loop(0, NUM_CHUNKS, chunk_body, 0)


def kernel(distances, distance_bias):
    d_flat = distances.reshape(N)
    bias_pad = jnp.zeros((LANES,), jnp.float32).at[: MAX_D + 1].set(distance_bias)
    out = _distance_bias_sc(d_flat, bias_pad)
    return out.reshape(B, S, S)


# speedup vs baseline: 507.9657x; 507.9657x over previous
import functools

import jax
import jax.numpy as jnp
from jax import lax
from jax.experimental import pallas as pl
from jax.experimental.pallas import tpu as pltpu
from jax.experimental.pallas import tpu_sc as plsc

MAX_D = 4
B, S = 4, 2048
N = B * S * S
NUM_WORKERS = 32
PER_WORKER = N // NUM_WORKERS
CHUNK = 16384
NUM_CHUNKS = PER_WORKER // CHUNK
LANES = 16

_mesh = plsc.VectorSubcoreMesh(core_axis_name="c", subcore_axis_name="s")


@functools.partial(
    pl.kernel,
    mesh=_mesh,
    out_type=jax.ShapeDtypeStruct((N,), jnp.float32),
    scratch_types=[
        pltpu.VMEM((LANES,), jnp.float32),
        pltpu.VMEM((CHUNK,), jnp.int32),
        pltpu.VMEM((CHUNK,), jnp.float32),
    ],
    compiler_params=pltpu.CompilerParams(needs_layout_passes=False),
)
def _distance_bias_sc(d_hbm, bias_hbm, out_hbm, tab_v, idx_v, out_v):
    wid = lax.axis_index("s") * 2 + lax.axis_index("c")
    pltpu.sync_copy(bias_hbm, tab_v)
    base = wid * PER_WORKER

    def chunk_body(ci, carry):
        off = base + ci * CHUNK
        pltpu.sync_copy(d_hbm.at[pl.ds(off, CHUNK)], idx_v)

        def vec_body(vi, c):
            idx = idx_v[pl.ds(vi * LANES, LANES)]
            idx = jnp.minimum(jnp.maximum(idx, 0), MAX_D)
            out_v[pl.ds(vi * LANES, LANES)] = plsc.load_gather(tab_v, [idx])
            return c

        lax.fori_loop(0, CHUNK // LANES, vec_body, 0)
        pltpu.sync_copy(out_v, out_hbm.at[pl.ds(off, CHUNK)])
        return carry

    lax.fori_loop(0, NUM_CHUNKS, chunk_body, 0)


def kernel(distances, distance_bias):
    d_flat = distances.reshape(N)
    bias_pad = jnp.zeros((LANES,), jnp.float32).at[: MAX_D + 1].set(distance_bias)
    out = _distance_bias_sc(d_flat, bias_pad)
    return out.reshape(B, S, S)


# speedup vs baseline: 700.7324x; 1.3795x over previous
import functools

import jax
import jax.numpy as jnp
from jax import lax
from jax.experimental import pallas as pl
from jax.experimental.pallas import tpu as pltpu
from jax.experimental.pallas import tpu_sc as plsc

MAX_D = 4
B, S = 4, 2048
N = B * S * S
NUM_WORKERS = 32
PER_WORKER = N // NUM_WORKERS
CHUNK = 16384
NUM_CHUNKS = PER_WORKER // CHUNK
LANES = 16
UNROLL = 8

_mesh = plsc.VectorSubcoreMesh(core_axis_name="c", subcore_axis_name="s")


@functools.partial(
    pl.kernel,
    mesh=_mesh,
    out_type=jax.ShapeDtypeStruct((N,), jnp.float32),
    scratch_types=[
        pltpu.VMEM((LANES,), jnp.float32),
        pltpu.VMEM((CHUNK,), jnp.int32),
        pltpu.VMEM((CHUNK,), jnp.float32),
    ],
    compiler_params=pltpu.CompilerParams(needs_layout_passes=False),
)
def _distance_bias_sc(d_hbm, bias_hbm, out_hbm, tab_v, idx_v, out_v):
    wid = lax.axis_index("s") * 2 + lax.axis_index("c")
    pltpu.sync_copy(bias_hbm, tab_v)
    base = wid * PER_WORKER

    def chunk_body(ci, carry):
        off = base + ci * CHUNK
        pltpu.sync_copy(d_hbm.at[pl.ds(off, CHUNK)], idx_v)

        def vec_body(vi, c):
            o = vi * (LANES * UNROLL)
            raw = [idx_v[pl.ds(o + u * LANES, LANES)] for u in range(UNROLL)]
            clamped = [
                plsc.bitcast(
                    jnp.minimum(plsc.bitcast(r, jnp.uint32), MAX_D), jnp.int32
                )
                for r in raw
            ]
            vals = [plsc.load_gather(tab_v, [c_]) for c_ in clamped]
            for u in range(UNROLL):
                out_v[pl.ds(o + u * LANES, LANES)] = vals[u]
            return c

        lax.fori_loop(0, CHUNK // (LANES * UNROLL), vec_body, 0)
        pltpu.sync_copy(out_v, out_hbm.at[pl.ds(off, CHUNK)])
        return carry

    lax.fori_loop(0, NUM_CHUNKS, chunk_body, 0)


def kernel(distances, distance_bias):
    d_flat = distances.reshape(N)
    bias_pad = jnp.zeros((LANES,), jnp.float32).at[: MAX_D + 1].set(distance_bias)
    out = _distance_bias_sc(d_flat, bias_pad)
    return out.reshape(B, S, S)


# speedup vs baseline: 894.4667x; 1.2765x over previous
import functools

import jax
import jax.numpy as jnp
from jax import lax
from jax.experimental import pallas as pl
from jax.experimental.pallas import tpu as pltpu
from jax.experimental.pallas import tpu_sc as plsc

MAX_D = 4
B, S = 4, 2048
N = B * S * S
NUM_WORKERS = 32
PER_WORKER = N // NUM_WORKERS
CHUNK = 16384
NUM_CHUNKS = PER_WORKER // CHUNK
LANES = 16
UNROLL = 8

_mesh = plsc.VectorSubcoreMesh(core_axis_name="c", subcore_axis_name="s")


@functools.partial(
    pl.kernel,
    mesh=_mesh,
    out_type=jax.ShapeDtypeStruct((N,), jnp.float32),
    scratch_types=[
        pltpu.VMEM((LANES,), jnp.float32),
        pltpu.VMEM((CHUNK,), jnp.int32),
        pltpu.VMEM((CHUNK,), jnp.int32),
        pltpu.VMEM((CHUNK,), jnp.float32),
        pltpu.VMEM((CHUNK,), jnp.float32),
        pltpu.SemaphoreType.DMA,
        pltpu.SemaphoreType.DMA,
        pltpu.SemaphoreType.DMA,
        pltpu.SemaphoreType.DMA,
    ],
    compiler_params=pltpu.CompilerParams(needs_layout_passes=False),
)
def _distance_bias_sc(
    d_hbm, bias_hbm, out_hbm,
    tab_v, idx0_v, idx1_v, out0_v, out1_v,
    sin0, sin1, sout0, sout1,
):
    wid = lax.axis_index("s") * 2 + lax.axis_index("c")
    pltpu.sync_copy(bias_hbm, tab_v)
    base = wid * PER_WORKER
    idx_bufs = (idx0_v, idx1_v)
    out_bufs = (out0_v, out1_v)
    in_sems = (sin0, sin1)
    out_sems = (sout0, sout1)

    def start_in(chunk, b):
        off = base + chunk * CHUNK
        pltpu.async_copy(d_hbm.at[pl.ds(off, CHUNK)], idx_bufs[b], in_sems[b])

    def wait_in(b):
        pltpu.make_async_copy(
            d_hbm.at[pl.ds(base, CHUNK)], idx_bufs[b], in_sems[b]
        ).wait()

    def start_out(chunk, b):
        off = base + chunk * CHUNK
        pltpu.async_copy(out_bufs[b], out_hbm.at[pl.ds(off, CHUNK)], out_sems[b])

    def wait_out(b):
        pltpu.make_async_copy(
            out_bufs[b], out_hbm.at[pl.ds(base, CHUNK)], out_sems[b]
        ).wait()

    def compute(b):
        idx_v, out_v = idx_bufs[b], out_bufs[b]

        def vec_body(vi, c):
            o = vi * (LANES * UNROLL)
            raw = [idx_v[pl.ds(o + u * LANES, LANES)] for u in range(UNROLL)]
            clamped = [
                plsc.bitcast(
                    jnp.minimum(plsc.bitcast(r, jnp.uint32), MAX_D), jnp.int32
                )
                for r in raw
            ]
            vals = [plsc.load_gather(tab_v, [c_]) for c_ in clamped]
            for u in range(UNROLL):
                out_v[pl.ds(o + u * LANES, LANES)] = vals[u]
            return c

        lax.fori_loop(0, CHUNK // (LANES * UNROLL), vec_body, 0)

    start_in(0, 0)
    start_in(1, 1)

    def pair_body(ci, carry):
        for b in range(2):
            chunk = ci * 2 + b
            wait_in(b)
            @pl.when(chunk >= 2)
            def _():
                wait_out(b)

            compute(b)
            start_out(chunk, b)

            @pl.when(chunk + 2 < NUM_CHUNKS)
            def _():
                start_in(chunk + 2, b)

        return carry

    lax.fori_loop(0, NUM_CHUNKS // 2, pair_body, 0)
    wait_out(0)
    wait_out(1)


def kernel(distances, distance_bias):
    d_flat = distances.reshape(N)
    bias_pad = jnp.zeros((LANES,), jnp.float32).at[: MAX_D + 1].set(distance_bias)
    out = _distance_bias_sc(d_flat, bias_pad)
    return out.reshape(B, S, S)


# speedup vs baseline: 2282.9889x; 2.5523x over previous
import functools

import jax
import jax.numpy as jnp
from jax import lax
from jax.experimental import pallas as pl
from jax.experimental.pallas import tpu as pltpu
from jax.experimental.pallas import tpu_sc as plsc

MAX_D = 4
B, S = 4, 2048
ROWS = B * S
NUM_WORKERS = 32
ROWS_PER_WORKER = ROWS // NUM_WORKERS
CHUNK_ROWS = 8
NUM_CHUNKS = ROWS_PER_WORKER // CHUNK_ROWS
LANES = 16
UNROLL = 8

_mesh = plsc.VectorSubcoreMesh(core_axis_name="c", subcore_axis_name="s")


@functools.partial(
    pl.kernel,
    mesh=_mesh,
    out_type=jax.ShapeDtypeStruct((ROWS, S), jnp.float32),
    scratch_types=[
        pltpu.VMEM((LANES,), jnp.float32),
        pltpu.VMEM((CHUNK_ROWS, S), jnp.int32),
        pltpu.VMEM((CHUNK_ROWS, S), jnp.int32),
        pltpu.VMEM((CHUNK_ROWS, S), jnp.float32),
        pltpu.VMEM((CHUNK_ROWS, S), jnp.float32),
        pltpu.SemaphoreType.DMA,
        pltpu.SemaphoreType.DMA,
        pltpu.SemaphoreType.DMA,
        pltpu.SemaphoreType.DMA,
    ],
    compiler_params=pltpu.CompilerParams(needs_layout_passes=False),
)
def _distance_bias_sc(
    d_hbm, bias_hbm, out_hbm,
    tab_v, idx0_v, idx1_v, out0_v, out1_v,
    sin0, sin1, sout0, sout1,
):
    wid = lax.axis_index("s") * 2 + lax.axis_index("c")
    pltpu.sync_copy(bias_hbm, tab_v)
    base_row = wid * ROWS_PER_WORKER
    idx_bufs = (idx0_v, idx1_v)
    out_bufs = (out0_v, out1_v)
    in_sems = (sin0, sin1)
    out_sems = (sout0, sout1)

    def start_in(chunk, b):
        r0 = base_row + chunk * CHUNK_ROWS
        pltpu.async_copy(d_hbm.at[pl.ds(r0, CHUNK_ROWS)], idx_bufs[b], in_sems[b])

    def wait_in(b):
        pltpu.make_async_copy(
            d_hbm.at[pl.ds(base_row, CHUNK_ROWS)], idx_bufs[b], in_sems[b]
        ).wait()

    def start_out(chunk, b):
        r0 = base_row + chunk * CHUNK_ROWS
        pltpu.async_copy(out_bufs[b], out_hbm.at[pl.ds(r0, CHUNK_ROWS)], out_sems[b])

    def wait_out(b):
        pltpu.make_async_copy(
            out_bufs[b], out_hbm.at[pl.ds(base_row, CHUNK_ROWS)], out_sems[b]
        ).wait()

    def compute(b):
        idx_v, out_v = idx_bufs[b], out_bufs[b]
        for r in range(CHUNK_ROWS):

            def vec_body(vi, c, r=r):
                o = vi * (LANES * UNROLL)
                raw = [idx_v[r, pl.ds(o + u * LANES, LANES)] for u in range(UNROLL)]
                clamped = [
                    plsc.bitcast(
                        jnp.minimum(plsc.bitcast(x, jnp.uint32), MAX_D), jnp.int32
                    )
                    for x in raw
                ]
                vals = [plsc.load_gather(tab_v, [c_]) for c_ in clamped]
                for u in range(UNROLL):
                    out_v[r, pl.ds(o + u * LANES, LANES)] = vals[u]
                return c

            lax.fori_loop(0, S // (LANES * UNROLL), vec_body, 0)

    start_in(0, 0)
    start_in(1, 1)

    def pair_body(ci, carry):
        for b in range(2):
            chunk = ci * 2 + b
            wait_in(b)
            @pl.when(chunk >= 2)
            def _():
                wait_out(b)

            compute(b)
            start_out(chunk, b)

            @pl.when(chunk + 2 < NUM_CHUNKS)
            def _():
                start_in(chunk + 2, b)

        return carry

    lax.fori_loop(0, NUM_CHUNKS // 2, pair_body, 0)
    wait_out(0)
    wait_out(1)


def kernel(distances, distance_bias):
    d2 = distances.reshape(ROWS, S)
    bias_pad = jnp.zeros((LANES,), jnp.float32).at[: MAX_D + 1].set(distance_bias)
    out = _distance_bias_sc(d2, bias_pad)
    return out.reshape(B, S, S)


# speedup vs baseline: 2441.3223x; 1.0694x over previous
import functools

import jax
import jax.numpy as jnp
from jax import lax
from jax.experimental import pallas as pl
from jax.experimental.pallas import tpu as pltpu
from jax.experimental.pallas import tpu_sc as plsc

MAX_D = 4
B, S = 4, 2048
ROWS = B * S
NUM_WORKERS = 32
ROWS_PER_WORKER = ROWS // NUM_WORKERS
CHUNK_ROWS = 4
NUM_CHUNKS = ROWS_PER_WORKER // CHUNK_ROWS
NBUF = 4
LANES = 16
UNROLL = 8

_mesh = plsc.VectorSubcoreMesh(core_axis_name="c", subcore_axis_name="s")


@functools.partial(
    pl.kernel,
    mesh=_mesh,
    out_type=jax.ShapeDtypeStruct((ROWS, S), jnp.float32),
    scratch_types=(
        [pltpu.VMEM((LANES,), jnp.float32)]
        + [pltpu.VMEM((CHUNK_ROWS, S), jnp.int32) for _ in range(NBUF)]
        + [pltpu.VMEM((CHUNK_ROWS, S), jnp.float32) for _ in range(NBUF)]
        + [pltpu.SemaphoreType.DMA for _ in range(2 * NBUF)]
    ),
    compiler_params=pltpu.CompilerParams(needs_layout_passes=False),
)
def _distance_bias_sc(d_hbm, bias_hbm, out_hbm, tab_v, *bufs_and_sems):
    idx_bufs = bufs_and_sems[:NBUF]
    out_bufs = bufs_and_sems[NBUF : 2 * NBUF]
    in_sems = bufs_and_sems[2 * NBUF : 3 * NBUF]
    out_sems = bufs_and_sems[3 * NBUF : 4 * NBUF]
    wid = lax.axis_index("s") * 2 + lax.axis_index("c")
    pltpu.sync_copy(bias_hbm, tab_v)
    base_row = wid * ROWS_PER_WORKER

    def start_in(chunk, b):
        r0 = base_row + chunk * CHUNK_ROWS
        pltpu.async_copy(d_hbm.at[pl.ds(r0, CHUNK_ROWS)], idx_bufs[b], in_sems[b])

    def wait_in(b):
        pltpu.make_async_copy(
            d_hbm.at[pl.ds(base_row, CHUNK_ROWS)], idx_bufs[b], in_sems[b]
        ).wait()

    def start_out(chunk, b):
        r0 = base_row + chunk * CHUNK_ROWS
        pltpu.async_copy(out_bufs[b], out_hbm.at[pl.ds(r0, CHUNK_ROWS)], out_sems[b])

    def wait_out(b):
        pltpu.make_async_copy(
            out_bufs[b], out_hbm.at[pl.ds(base_row, CHUNK_ROWS)], out_sems[b]
        ).wait()

    def compute(b):
        idx_v, out_v = idx_bufs[b], out_bufs[b]
        for r in range(CHUNK_ROWS):

            def vec_body(vi, c, r=r):
                o = vi * (LANES * UNROLL)
                raw = [idx_v[r, pl.ds(o + u * LANES, LANES)] for u in range(UNROLL)]
                clamped = [
                    plsc.bitcast(
                        jnp.minimum(plsc.bitcast(x, jnp.uint32), MAX_D), jnp.int32
                    )
                    for x in raw
                ]
                vals = [plsc.load_gather(tab_v, [c_]) for c_ in clamped]
                for u in range(UNROLL):
                    out_v[r, pl.ds(o + u * LANES, LANES)] = vals[u]
                return c

            lax.fori_loop(0, S // (LANES * UNROLL), vec_body, 0)

    for b in range(NBUF):
        start_in(b, b)

    def ring_body(ci, carry):
        for b in range(NBUF):
            chunk = ci * NBUF + b
            wait_in(b)
            @pl.when(chunk >= NBUF)
            def _():
                wait_out(b)

            compute(b)
            start_out(chunk, b)

            @pl.when(chunk + NBUF < NUM_CHUNKS)
            def _():
                start_in(chunk + NBUF, b)

        return carry

    lax.fori_loop(0, NUM_CHUNKS // NBUF, ring_body, 0)
    for b in range(NBUF):
        wait_out(b)


def kernel(distances, distance_bias):
    d2 = distances.reshape(ROWS, S)
    bias_pad = jnp.zeros((LANES,), jnp.float32).at[: MAX_D + 1].set(distance_bias)
    out = _distance_bias_sc(d2, bias_pad)
    return out.reshape(B, S, S)


# speedup vs baseline: 2497.0936x; 1.0228x over previous
import functools

import jax
import jax.numpy as jnp
from jax import lax
from jax.experimental import pallas as pl
from jax.experimental.pallas import tpu as pltpu
from jax.experimental.pallas import tpu_sc as plsc

MAX_D = 4
B, S = 4, 2048
ROWS = B * S
NUM_WORKERS = 32
ROWS_PER_WORKER = ROWS // NUM_WORKERS
CHUNK_ROWS = 4
NUM_CHUNKS = ROWS_PER_WORKER // CHUNK_ROWS
NBUF = 4
LANES = 16
UNROLL = 8

_mesh = plsc.VectorSubcoreMesh(core_axis_name="c", subcore_axis_name="s")


@functools.partial(
    pl.kernel,
    mesh=_mesh,
    out_type=jax.ShapeDtypeStruct((ROWS, S), jnp.float32),
    scratch_types=(
        [pltpu.VMEM((LANES,), jnp.float32)]
        + [pltpu.VMEM((CHUNK_ROWS, S), jnp.int32) for _ in range(NBUF)]
        + [pltpu.VMEM((CHUNK_ROWS, S), jnp.float32) for _ in range(NBUF)]
        + [pltpu.SemaphoreType.DMA for _ in range(2 * NBUF)]
    ),
    compiler_params=pltpu.CompilerParams(needs_layout_passes=False),
)
def _distance_bias_sc(d_hbm, bias_hbm, out_hbm, tab_v, *bufs_and_sems):
    idx_bufs = bufs_and_sems[:NBUF]
    out_bufs = bufs_and_sems[NBUF : 2 * NBUF]
    in_sems = bufs_and_sems[2 * NBUF : 3 * NBUF]
    out_sems = bufs_and_sems[3 * NBUF : 4 * NBUF]
    wid = lax.axis_index("s") * 2 + lax.axis_index("c")
    pltpu.sync_copy(bias_hbm, tab_v)
    base_row = wid * ROWS_PER_WORKER
    tab_vec = tab_v[pl.ds(0, LANES)]
    bsplat = [
        jax.lax.broadcast_in_dim(tab_vec[k], (LANES,), ()) for k in range(MAX_D + 1)
    ]

    def start_in(chunk, b):
        r0 = base_row + chunk * CHUNK_ROWS
        pltpu.async_copy(d_hbm.at[pl.ds(r0, CHUNK_ROWS)], idx_bufs[b], in_sems[b])

    def wait_in(b):
        pltpu.make_async_copy(
            d_hbm.at[pl.ds(base_row, CHUNK_ROWS)], idx_bufs[b], in_sems[b]
        ).wait()

    def start_out(chunk, b):
        r0 = base_row + chunk * CHUNK_ROWS
        pltpu.async_copy(out_bufs[b], out_hbm.at[pl.ds(r0, CHUNK_ROWS)], out_sems[b])

    def wait_out(b):
        pltpu.make_async_copy(
            out_bufs[b], out_hbm.at[pl.ds(base_row, CHUNK_ROWS)], out_sems[b]
        ).wait()

    def compute(b):
        idx_v, out_v = idx_bufs[b], out_bufs[b]
        for r in range(CHUNK_ROWS):

            def vec_body(vi, c, r=r):
                o = vi * (LANES * UNROLL)
                raw = [idx_v[r, pl.ds(o + u * LANES, LANES)] for u in range(UNROLL)]
                vals = []
                for u, x in enumerate(raw):
                    if u % 2 == 0:
                        cl = plsc.bitcast(
                            jnp.minimum(plsc.bitcast(x, jnp.uint32), MAX_D),
                            jnp.int32,
                        )
                        vals.append(plsc.load_gather(tab_v, [cl]))
                    else:
                        v = jnp.where(x >= 1, bsplat[1], bsplat[0])
                        v = jnp.where(x >= 2, bsplat[2], v)
                        v = jnp.where(x >= 3, bsplat[3], v)
                        v = jnp.where(x >= 4, bsplat[4], v)
                        vals.append(v)
                for u in range(UNROLL):
                    out_v[r, pl.ds(o + u * LANES, LANES)] = vals[u]
                return c

            lax.fori_loop(0, S // (LANES * UNROLL), vec_body, 0)

    for b in range(NBUF):
        start_in(b, b)

    def ring_body(ci, carry):
        for b in range(NBUF):
            chunk = ci * NBUF + b
            wait_in(b)
            @pl.when(chunk >= NBUF)
            def _():
                wait_out(b)

            compute(b)
            start_out(chunk, b)

            @pl.when(chunk + NBUF < NUM_CHUNKS)
            def _():
                start_in(chunk + NBUF, b)

        return carry

    lax.fori_loop(0, NUM_CHUNKS // NBUF, ring_body, 0)
    for b in range(NBUF):
        wait_out(b)


def kernel(distances, distance_bias):
    d2 = distances.reshape(ROWS, S)
    bias_pad = jnp.zeros((LANES,), jnp.float32).at[: MAX_D + 1].set(distance_bias)
    out = _distance_bias_sc(d2, bias_pad)
    return out.reshape(B, S, S)
